# Initial kernel scaffold; baseline (speedup 1.0000x reference)
import jax
import jax.numpy as jnp
from jax.experimental import pallas as pl


def kernel(x, edge_index, edge_attr, params):
    raise NotImplementedError("write your pallas kernel here")


# speedup vs baseline: 26.9673x; 26.9673x over previous
import functools
import jax
import jax.numpy as jnp
from jax import lax
from jax.experimental import pallas as pl
from jax.experimental.pallas import tpu as pltpu, tpu_sc as plsc

_N = 10000
_E = 320000
_NC = 2
_NS = 16
_NW = _NC * _NS
_EW = _E // _NW
_NTILE = _N // _NS

_HEADS = (4, 4, 4, 1, 1, 1)
_OFFS = (0, 4, 8, 12, 13, 14)


def _iota16():
  return lax.iota(jnp.int32, 16)


def _full16(v):
  return jnp.full((16,), v, dtype=jnp.int32)


def _lrelu_exp(z):
  return jnp.exp(jnp.where(z > 0, z, 0.2 * z))


def _tc_edge_pre(ea_ref, wc_ref, out_ref):
  v = jnp.dot(ea_ref[...], wc_ref[...], preferred_element_type=jnp.float32)
  one15 = (lax.broadcasted_iota(jnp.int32, (1, 16), 1) == 15
           ).astype(jnp.float32)
  out_ref[...] = v + one15


def _run_edge_pre(edge_attr, wcomb):
  blk = 2000
  grid = _E // blk
  return pl.pallas_call(
      _tc_edge_pre,
      grid=(grid,),
      in_specs=[
          pl.BlockSpec((blk, 16), lambda i: (i, 0)),
          pl.BlockSpec((16, 16), lambda i: (0, 0)),
      ],
      out_specs=pl.BlockSpec((blk, 16), lambda i: (i, 0)),
      out_shape=jax.ShapeDtypeStruct((_E, 16), jnp.float32),
  )(edge_attr, wcomb)


def _tc_d0(x_ref, w0_ref, lacc_ref, xl_ref, le_ref):
  xl_ref[...] = jnp.dot(x_ref[...], w0_ref[...],
                        preferred_element_type=jnp.float32)
  s = lacc_ref[0] + lacc_ref[1]
  cnt = s[:, 15:16]
  le_ref[...] = s / jnp.maximum(cnt, 1.0)


def _run_d0(x, w0, lacc):
  return pl.pallas_call(
      _tc_d0,
      out_shape=(
          jax.ShapeDtypeStruct((_N, 4), jnp.float32),
          jax.ShapeDtypeStruct((_N, 16), jnp.float32),
      ),
  )(x, w0, lacc)


def _tc_mid(i, heads, hc_next, acc_ref, xl_ref, le_ref, asd_ref, wn_ref,
            bb_ref, out_ref):
  h = heads
  xl = xl_ref[...]
  a_s = asd_ref[0:1, :h]
  a_d = asd_ref[1:2, :h]
  z = xl[:, :h] * a_s + xl[:, :h] * a_d \
      + le_ref[:, _OFFS[i]:_OFFS[i] + h]
  wl = _lrelu_exp(z)
  s = acc_ref[0] + acc_ref[1]
  denom = s[:, 0:h] + wl
  num = s[:, 4:4 + h] + wl * xl[:, :h]
  o = num / (denom + 1e-16) + bb_ref[0:1, :h]
  o = jnp.maximum(o, 0.0)
  if i < 3:
    mu = jnp.mean(o, axis=0, keepdims=True)
    var = jnp.mean((o - mu) * (o - mu), axis=0, keepdims=True)
    o = (o - mu) / jnp.sqrt(var + 1e-5) * bb_ref[1:2, :h] + bb_ref[2:3, :h]
  if h == 1:
    out_ref[...] = o * wn_ref[0:1, :]
  else:
    out_ref[...] = jnp.dot(o, wn_ref[...],
                           preferred_element_type=jnp.float32)


def _run_mid(i, acc, xl, le, a_src, a_dst, w_next, bias, gamma, beta):
  heads = _HEADS[i]
  hc_next = w_next.shape[1]
  asd = jnp.zeros((2, 4), jnp.float32)
  asd = asd.at[0, :heads].set(a_src)
  asd = asd.at[1, :heads].set(a_dst)
  bb = jnp.zeros((3, 4), jnp.float32)
  bb = bb.at[0, :heads].set(bias)
  if gamma is not None:
    bb = bb.at[1, :heads].set(gamma)
    bb = bb.at[2, :heads].set(beta)
  body = functools.partial(_tc_mid, i, heads, hc_next)
  return pl.pallas_call(
      body,
      out_shape=jax.ShapeDtypeStruct((_N, hc_next), jnp.float32),
  )(acc, xl, le, asd, w_next, bb)


def _tc_d5in(i, acc_ref, xl_ref, le_ref, asd_ref, w5_ref, bb_ref, sc_ref,
             t5_ref, ad8_ref):
  xl = xl_ref[...]
  z = xl * asd_ref[0:1, :1] + xl * asd_ref[1:2, :1] \
      + le_ref[:, _OFFS[4]:_OFFS[4] + 1]
  wl = _lrelu_exp(z)
  s = acc_ref[0] + acc_ref[1]
  denom = s[:, 0:1] + wl
  num = s[:, 4:5] + wl * xl
  o = num / (denom + 1e-16) + bb_ref[0:1, :1]
  o = jnp.maximum(o, 0.0)
  xl5 = o * w5_ref[...]
  as5 = o * sc_ref[0, 0]
  ad5 = o * sc_ref[0, 1]
  nb = xl.shape[0]
  t5_ref[:, 0:128] = xl5
  t5_ref[:, 128:129] = as5
  t5_ref[:, 129:136] = jnp.zeros((nb, 7), jnp.float32)
  ad8_ref[:, 0:1] = ad5
  ad8_ref[:, 1:8] = jnp.zeros((nb, 7), jnp.float32)


def _run_d5in(acc, xl, le, a_src, a_dst, w5, bias, sa5, sd5):
  asd = jnp.zeros((2, 4), jnp.float32)
  asd = asd.at[0, 0].set(a_src[0])
  asd = asd.at[1, 0].set(a_dst[0])
  bb = jnp.zeros((3, 4), jnp.float32)
  bb = bb.at[0, 0].set(bias[0])
  sc = jnp.stack([sa5, sd5]).reshape(1, 2)
  body = functools.partial(_tc_d5in, 4)
  blk = 2000
  return pl.pallas_call(
      body,
      grid=(_N // blk,),
      in_specs=[
          pl.BlockSpec((2, blk, 8), lambda i: (0, i, 0)),
          pl.BlockSpec((blk, 1), lambda i: (i, 0)),
          pl.BlockSpec((blk, 16), lambda i: (i, 0)),
          pl.BlockSpec((2, 4), lambda i: (0, 0)),
          pl.BlockSpec((1, 128), lambda i: (0, 0)),
          pl.BlockSpec((3, 4), lambda i: (0, 0)),
          pl.BlockSpec((1, 2), lambda i: (0, 0)),
      ],
      out_specs=(
          pl.BlockSpec((blk, 136), lambda i: (i, 0)),
          pl.BlockSpec((blk, 8), lambda i: (i, 0)),
      ),
      out_shape=(
          jax.ShapeDtypeStruct((_N, 136), jnp.float32),
          jax.ShapeDtypeStruct((_N, 8), jnp.float32),
      ),
  )(acc, xl, le, asd, w5, bb, sc)


def _tc_final(acc_ref, t5_ref, ad8_ref, le_ref, b5_ref, out_ref):
  s = acc_ref[0] + acc_ref[1]
  xl5 = t5_ref[:, 0:128]
  z = t5_ref[:, 128:129] + ad8_ref[:, 0:1] + le_ref[:, 14:15]
  wl = _lrelu_exp(z)
  denom = s[:, 128:129] + wl
  num = s[:, 0:128] + wl * xl5
  out_ref[...] = num / (denom + 1e-16) + b5_ref[...]


def _run_final(acc, t5, ad8, le, bias5):
  blk = 2000
  return pl.pallas_call(
      _tc_final,
      grid=(_N // blk,),
      in_specs=[
          pl.BlockSpec((2, blk, 136), lambda i: (0, i, 0)),
          pl.BlockSpec((blk, 136), lambda i: (i, 0)),
          pl.BlockSpec((blk, 8), lambda i: (i, 0)),
          pl.BlockSpec((blk, 16), lambda i: (i, 0)),
          pl.BlockSpec((1, 128), lambda i: (0, 0)),
      ],
      out_specs=pl.BlockSpec((blk, 128), lambda i: (i, 0)),
      out_shape=jax.ShapeDtypeStruct((_N, 128), jnp.float32),
  )(acc, t5, ad8, le, bias5.reshape(1, 128))


def _mesh():
  return plsc.VectorSubcoreMesh(core_axis_name="c", subcore_axis_name="s")


def _zero_vmem(ref, rows, cols):
  z = jnp.zeros((16,), jnp.float32)
  def body(j, _):
    flat = j * 16 + _iota16()
    plsc.store_scatter(ref, [flat // cols, flat % cols], z)
    return 0
  lax.fori_loop(0, rows * cols // 16, body, 0)


_SLAB = 624


def _acc_init_and_barrier(acc, zsrc8):
  s = lax.axis_index("s")
  def body(q, _):
    pltpu.sync_copy(zsrc8, acc.at[pl.ds(s * _SLAB + q * 8, 8)])
    return 0
  lax.fori_loop(0, _SLAB // 8, body, 0)
  @pl.when(s == _NS - 1)
  def _():
    pltpu.sync_copy(zsrc8, acc.at[pl.ds(_SLAB * _NS, 8)])
    pltpu.sync_copy(zsrc8, acc.at[pl.ds(_SLAB * _NS + 8, 8)])
  plsc.subcore_barrier()


def _acc_writeback(acc, out_hbm, c):
  s = lax.axis_index("s")
  plsc.subcore_barrier()
  pltpu.sync_copy(acc.at[pl.ds(s * _SLAB, _SLAB)],
                  out_hbm.at[c, pl.ds(s * _SLAB, _SLAB)])
  @pl.when(s == _NS - 1)
  def _():
    pltpu.sync_copy(acc.at[pl.ds(_SLAB * _NS, 16)],
                    out_hbm.at[c, pl.ds(_SLAB * _NS, 16)])


def _sc_pre_body(vals_hbm, dst_hbm, out_hbm, vbuf, dbuf, acc):
  CH = 400
  c = lax.axis_index("c")
  s = lax.axis_index("s")
  g = c * _NS + s
  _zero_vmem(vbuf, 125, 16)
  _acc_init_and_barrier(acc, vbuf.at[pl.ds(0, 8)])
  def chunk(k, _):
    base = g * _EW + k * CH
    pltpu.sync_copy(vals_hbm.at[pl.ds(base, CH)], vbuf)
    pltpu.sync_copy(dst_hbm.at[pl.ds(base, CH)], dbuf)
    pltpu.sync_copy(vbuf, acc.at[dbuf], add=True)
    return 0
  lax.fori_loop(0, _EW // CH, chunk, 0)
  _acc_writeback(acc, out_hbm, c)


def _run_sc_pre(ealpha16, dst):
  k = pl.kernel(
      _sc_pre_body,
      out_type=jax.ShapeDtypeStruct((_NC, _N, 16), jnp.float32),
      mesh=_mesh(),
      compiler_params=pltpu.CompilerParams(needs_layout_passes=False, use_tc_tiling_on_sc=False),
      scratch_types=[
          pltpu.VMEM((400, 16), jnp.float32),
          pltpu.VMEM((400,), jnp.int32),
          pltpu.VMEM_SHARED((_N, 16), jnp.float32),
      ],
  )
  return k(ealpha16, dst)


def _sc_small_body(heads, wt, off, xl_hbm, src_hbm, dst_hbm, eal_hbm, att_hbm,
                   out_hbm, xlt, sbuf, dbuf, ebuf, stage, attv, acc):
  CH = 400
  c = lax.axis_index("c")
  s = lax.axis_index("s")
  g = c * _NS + s
  _zero_vmem(stage, CH, 8)
  _acc_init_and_barrier(acc, stage.at[pl.ds(0, 8)])
  pltpu.sync_copy(xl_hbm, xlt)
  pltpu.sync_copy(att_hbm, attv)
  asc = [plsc.load_gather(attv, [_full16(hh)]) for hh in range(heads)]
  adc = [plsc.load_gather(attv, [_full16(8 + hh)]) for hh in range(heads)]

  def chunk(k, _):
    base = g * _EW + k * CH
    pltpu.sync_copy(src_hbm.at[pl.ds(base, CH)], sbuf)
    pltpu.sync_copy(dst_hbm.at[pl.ds(base, CH)], dbuf)
    pltpu.sync_copy(eal_hbm.at[pl.ds(base, CH)], ebuf)
    def grp(j, _):
      l16 = j * 16 + _iota16()
      s16 = sbuf[pl.ds(j * 16, 16)]
      d16 = dbuf[pl.ds(j * 16, 16)]
      for hh in range(heads):
        gs = plsc.load_gather(xlt, [s16 * wt + hh])
        gd = plsc.load_gather(xlt, [d16 * wt + hh])
        ge = plsc.load_gather(ebuf, [l16, _full16(off + hh)])
        w = _lrelu_exp(gs * asc[hh] + gd * adc[hh] + ge)
        plsc.store_scatter(stage, [l16, _full16(hh)], w)
        plsc.store_scatter(stage, [l16, _full16(4 + hh)], w * gs)
      return 0
    lax.fori_loop(0, CH // 16, grp, 0)
    pltpu.sync_copy(stage, acc.at[dbuf], add=True)
    return 0
  lax.fori_loop(0, _EW // CH, chunk, 0)
  _acc_writeback(acc, out_hbm, c)


def _run_sc_small(i, xl, src, dst, ealpha16, a_src, a_dst):
  heads = _HEADS[i]
  wt = xl.shape[1]
  att = jnp.zeros((16,), jnp.float32)
  att = att.at[:heads].set(a_src)
  att = att.at[8:8 + heads].set(a_dst)
  body = functools.partial(_sc_small_body, heads, wt, _OFFS[i])
  k = pl.kernel(
      body,
      out_type=jax.ShapeDtypeStruct((_NC, _N, 8), jnp.float32),
      mesh=_mesh(),
      compiler_params=pltpu.CompilerParams(needs_layout_passes=False, use_tc_tiling_on_sc=False),
      scratch_types=[
          pltpu.VMEM((_N * wt,), jnp.float32),
          pltpu.VMEM((400,), jnp.int32),
          pltpu.VMEM((400,), jnp.int32),
          pltpu.VMEM((400, 16), jnp.float32),
          pltpu.VMEM((400, 8), jnp.float32),
          pltpu.VMEM((16,), jnp.float32),
          pltpu.VMEM_SHARED((_N, 8), jnp.float32),
      ],
  )
  return k(xl.reshape(-1), src, dst, ealpha16, att)


def _sc_l5_body(t5_hbm, ad8_hbm, src_hbm, dst_hbm, eal_hbm, out_hbm,
                sbuf, dbuf, ebuf, rows, drows, stage, wbuf, acc, sem):
  CH = 80
  c = lax.axis_index("c")
  s = lax.axis_index("s")
  g = c * _NS + s
  _zero_vmem(stage, CH, 136)
  _acc_init_and_barrier(acc, stage.at[pl.ds(0, 8)])

  def chunk(k, _):
    base = g * _EW + k * CH
    pltpu.sync_copy(src_hbm.at[pl.ds(base, CH)], sbuf)
    pltpu.sync_copy(dst_hbm.at[pl.ds(base, CH)], dbuf)
    pltpu.sync_copy(eal_hbm.at[pl.ds(base, CH)], ebuf)
    pltpu.async_copy(t5_hbm.at[sbuf], rows, sem).wait()
    pltpu.async_copy(ad8_hbm.at[dbuf], drows, sem).wait()
    def grp(j, _):
      l16 = j * 16 + _iota16()
      a_s = plsc.load_gather(rows, [l16, _full16(128)])
      a_d = plsc.load_gather(drows, [l16, _full16(0)])
      ge = plsc.load_gather(ebuf, [l16, _full16(14)])
      w = _lrelu_exp(a_s + a_d + ge)
      wbuf[pl.ds(j * 16, 16)] = w
      plsc.store_scatter(stage, [l16, _full16(128)], w)
      return 0
    lax.fori_loop(0, CH // 16, grp, 0)
    def edge(e, _):
      w16 = plsc.load_gather(wbuf, [_full16(e)])
      e16 = _full16(e)
      for kc in range(8):
        col = kc * 16 + _iota16()
        v = plsc.load_gather(rows, [e16, col])
        plsc.store_scatter(stage, [e16, col], w16 * v)
      return 0
    lax.fori_loop(0, CH, edge, 0)
    pltpu.sync_copy(stage, acc.at[dbuf], add=True)
    return 0
  lax.fori_loop(0, _EW // CH, chunk, 0)
  _acc_writeback(acc, out_hbm, c)


def _run_sc_l5(t5, ad8, src, dst, ealpha16):
  k = pl.kernel(
      _sc_l5_body,
      out_type=jax.ShapeDtypeStruct((_NC, _N, 136), jnp.float32),
      mesh=_mesh(),
      compiler_params=pltpu.CompilerParams(needs_layout_passes=False, use_tc_tiling_on_sc=False),
      scratch_types=[
          pltpu.VMEM((80,), jnp.int32),
          pltpu.VMEM((80,), jnp.int32),
          pltpu.VMEM((80, 16), jnp.float32),
          pltpu.VMEM((80, 136), jnp.float32),
          pltpu.VMEM((80, 8), jnp.float32),
          pltpu.VMEM((80, 136), jnp.float32),
          pltpu.VMEM((80,), jnp.float32),
          pltpu.VMEM_SHARED((_N, 136), jnp.float32),
          pltpu.SemaphoreType.DMA,
      ],
  )
  return k(t5, ad8, src, dst, ealpha16)


def kernel(x, edge_index, edge_attr, params):
  gats = params["gats"]
  bns = params["bns"]
  src = edge_index[0]
  dst = edge_index[1]

  chs = (1, 1, 1, 1, 1, 128)
  wcols = []
  for l in range(6):
    p = gats[l]
    h, ch = _HEADS[l], chs[l]
    wer = p["W_e"].reshape(16, h, ch)
    wcols.append(jnp.einsum("dhc,hc->dh", wer, p["att_edge"]))
  wcomb = jnp.concatenate(wcols, axis=1)
  wcomb = jnp.concatenate([wcomb, jnp.zeros((16, 1), jnp.float32)], axis=1)

  ealpha16 = _run_edge_pre(edge_attr, wcomb)
  lacc = _run_sc_pre(ealpha16, dst)
  xl, le = _run_d0(x, gats[0]["W"], lacc)

  for i in range(5):
    p = gats[i]
    a_src = p["att_src"][:, 0]
    a_dst = p["att_dst"][:, 0]
    acc = _run_sc_small(i, xl, src, dst, ealpha16, a_src, a_dst)
    if i < 4:
      xl = _run_mid(i, acc, xl, le, a_src, a_dst, gats[i + 1]["W"],
                    p["bias"],
                    bns[i]["gamma"] if i < 3 else None,
                    bns[i]["beta"] if i < 3 else None)
    else:
      p5 = gats[5]
      sa5 = jnp.sum(p5["W"][0] * p5["att_src"][0])
      sd5 = jnp.sum(p5["W"][0] * p5["att_dst"][0])
      t5, ad8 = _run_d5in(acc, xl, le, a_src, a_dst, p5["W"], p["bias"],
                          sa5, sd5)

  acc5 = _run_sc_l5(t5, ad8, src, dst, ealpha16)
  return _run_final(acc5, t5, ad8, le, gats[5]["bias"])


# speedup vs baseline: 27.4576x; 1.0182x over previous
import functools
import jax
import jax.numpy as jnp
from jax import lax
from jax.experimental import pallas as pl
from jax.experimental.pallas import tpu as pltpu, tpu_sc as plsc

_N = 10000
_E = 320000
_NC = 2
_NS = 16
_NW = _NC * _NS
_EW = _E // _NW
_NTILE = _N // _NS

_HEADS = (4, 4, 4, 1, 1, 1)
_OFFS = (0, 4, 8, 12, 13, 14)


def _iota16():
  return lax.iota(jnp.int32, 16)


def _full16(v):
  return jnp.full((16,), v, dtype=jnp.int32)


def _lrelu_exp(z):
  return jnp.exp(jnp.where(z > 0, z, 0.2 * z))


def _tc_edge_pre(ea_ref, wc_ref, out_ref):
  v = jnp.dot(ea_ref[...], wc_ref[...], preferred_element_type=jnp.float32)
  one15 = (lax.broadcasted_iota(jnp.int32, (1, 16), 1) == 15
           ).astype(jnp.float32)
  out_ref[...] = v + one15


def _run_edge_pre(edge_attr, wcomb):
  blk = 2000
  grid = _E // blk
  return pl.pallas_call(
      _tc_edge_pre,
      grid=(grid,),
      in_specs=[
          pl.BlockSpec((blk, 16), lambda i: (i, 0)),
          pl.BlockSpec((16, 16), lambda i: (0, 0)),
      ],
      out_specs=pl.BlockSpec((blk, 16), lambda i: (i, 0)),
      out_shape=jax.ShapeDtypeStruct((_E, 16), jnp.float32),
  )(edge_attr, wcomb)


def _tc_d0(x_ref, w0_ref, lacc_ref, xl_ref, le_ref):
  xl_ref[...] = jnp.dot(x_ref[...], w0_ref[...],
                        preferred_element_type=jnp.float32)
  s = lacc_ref[0] + lacc_ref[1]
  cnt = s[:, 15:16]
  le_ref[...] = s / jnp.maximum(cnt, 1.0)


def _run_d0(x, w0, lacc):
  return pl.pallas_call(
      _tc_d0,
      out_shape=(
          jax.ShapeDtypeStruct((_N, 4), jnp.float32),
          jax.ShapeDtypeStruct((_N, 16), jnp.float32),
      ),
  )(x, w0, lacc)


def _tc_mid(i, heads, hc_next, acc_ref, xl_ref, le_ref, asd_ref, wn_ref,
            bb_ref, out_ref):
  h = heads
  xl = xl_ref[...]
  a_s = asd_ref[0:1, :h]
  a_d = asd_ref[1:2, :h]
  z = xl[:, :h] * a_s + xl[:, :h] * a_d \
      + le_ref[:, _OFFS[i]:_OFFS[i] + h]
  wl = _lrelu_exp(z)
  s = acc_ref[0] + acc_ref[1]
  denom = s[:, 0:h] + wl
  num = s[:, 4:4 + h] + wl * xl[:, :h]
  o = num / (denom + 1e-16) + bb_ref[0:1, :h]
  o = jnp.maximum(o, 0.0)
  if i < 3:
    mu = jnp.mean(o, axis=0, keepdims=True)
    var = jnp.mean((o - mu) * (o - mu), axis=0, keepdims=True)
    o = (o - mu) / jnp.sqrt(var + 1e-5) * bb_ref[1:2, :h] + bb_ref[2:3, :h]
  if h == 1:
    out_ref[...] = o * wn_ref[0:1, :]
  else:
    out_ref[...] = jnp.dot(o, wn_ref[...],
                           preferred_element_type=jnp.float32)


def _run_mid(i, acc, xl, le, a_src, a_dst, w_next, bias, gamma, beta):
  heads = _HEADS[i]
  hc_next = w_next.shape[1]
  asd = jnp.zeros((2, 4), jnp.float32)
  asd = asd.at[0, :heads].set(a_src)
  asd = asd.at[1, :heads].set(a_dst)
  bb = jnp.zeros((3, 4), jnp.float32)
  bb = bb.at[0, :heads].set(bias)
  if gamma is not None:
    bb = bb.at[1, :heads].set(gamma)
    bb = bb.at[2, :heads].set(beta)
  body = functools.partial(_tc_mid, i, heads, hc_next)
  return pl.pallas_call(
      body,
      out_shape=jax.ShapeDtypeStruct((_N, hc_next), jnp.float32),
  )(acc, xl, le, asd, w_next, bb)


def _tc_d5in(i, acc_ref, xl_ref, le_ref, asd_ref, w5_ref, bb_ref, sc_ref,
             t5_ref, ad8_ref):
  xl = xl_ref[...]
  z = xl * asd_ref[0:1, :1] + xl * asd_ref[1:2, :1] \
      + le_ref[:, _OFFS[4]:_OFFS[4] + 1]
  wl = _lrelu_exp(z)
  s = acc_ref[0] + acc_ref[1]
  denom = s[:, 0:1] + wl
  num = s[:, 4:5] + wl * xl
  o = num / (denom + 1e-16) + bb_ref[0:1, :1]
  o = jnp.maximum(o, 0.0)
  xl5 = o * w5_ref[...]
  as5 = o * sc_ref[0, 0]
  ad5 = o * sc_ref[0, 1]
  nb = xl.shape[0]
  t5_ref[:, 0:128] = xl5
  t5_ref[:, 128:129] = as5
  t5_ref[:, 129:136] = jnp.zeros((nb, 7), jnp.float32)
  ad8_ref[:, 0:1] = ad5
  ad8_ref[:, 1:8] = jnp.zeros((nb, 7), jnp.float32)


def _run_d5in(acc, xl, le, a_src, a_dst, w5, bias, sa5, sd5):
  asd = jnp.zeros((2, 4), jnp.float32)
  asd = asd.at[0, 0].set(a_src[0])
  asd = asd.at[1, 0].set(a_dst[0])
  bb = jnp.zeros((3, 4), jnp.float32)
  bb = bb.at[0, 0].set(bias[0])
  sc = jnp.stack([sa5, sd5]).reshape(1, 2)
  body = functools.partial(_tc_d5in, 4)
  blk = 2000
  return pl.pallas_call(
      body,
      grid=(_N // blk,),
      in_specs=[
          pl.BlockSpec((2, blk, 8), lambda i: (0, i, 0)),
          pl.BlockSpec((blk, 1), lambda i: (i, 0)),
          pl.BlockSpec((blk, 16), lambda i: (i, 0)),
          pl.BlockSpec((2, 4), lambda i: (0, 0)),
          pl.BlockSpec((1, 128), lambda i: (0, 0)),
          pl.BlockSpec((3, 4), lambda i: (0, 0)),
          pl.BlockSpec((1, 2), lambda i: (0, 0)),
      ],
      out_specs=(
          pl.BlockSpec((blk, 136), lambda i: (i, 0)),
          pl.BlockSpec((blk, 8), lambda i: (i, 0)),
      ),
      out_shape=(
          jax.ShapeDtypeStruct((_N, 136), jnp.float32),
          jax.ShapeDtypeStruct((_N, 8), jnp.float32),
      ),
  )(acc, xl, le, asd, w5, bb, sc)


def _tc_final(acc_ref, t5_ref, ad8_ref, le_ref, b5_ref, out_ref):
  s = acc_ref[0] + acc_ref[1]
  xl5 = t5_ref[:, 0:128]
  z = t5_ref[:, 128:129] + ad8_ref[:, 0:1] + le_ref[:, 14:15]
  wl = _lrelu_exp(z)
  denom = s[:, 128:129] + wl
  num = s[:, 0:128] + wl * xl5
  out_ref[...] = num / (denom + 1e-16) + b5_ref[...]


def _run_final(acc, t5, ad8, le, bias5):
  blk = 2000
  return pl.pallas_call(
      _tc_final,
      grid=(_N // blk,),
      in_specs=[
          pl.BlockSpec((2, blk, 136), lambda i: (0, i, 0)),
          pl.BlockSpec((blk, 136), lambda i: (i, 0)),
          pl.BlockSpec((blk, 8), lambda i: (i, 0)),
          pl.BlockSpec((blk, 16), lambda i: (i, 0)),
          pl.BlockSpec((1, 128), lambda i: (0, 0)),
      ],
      out_specs=pl.BlockSpec((blk, 128), lambda i: (i, 0)),
      out_shape=jax.ShapeDtypeStruct((_N, 128), jnp.float32),
  )(acc, t5, ad8, le, bias5.reshape(1, 128))


def _mesh():
  return plsc.VectorSubcoreMesh(core_axis_name="c", subcore_axis_name="s")


def _zero_vmem(ref, rows, cols):
  z = jnp.zeros((16,), jnp.float32)
  def body(j, _):
    flat = j * 16 + _iota16()
    plsc.store_scatter(ref, [flat // cols, flat % cols], z)
    return 0
  lax.fori_loop(0, rows * cols // 16, body, 0)


_SLAB = 624


def _acc_init_and_barrier(acc, zsrc8):
  s = lax.axis_index("s")
  def body(q, _):
    pltpu.sync_copy(zsrc8, acc.at[pl.ds(s * _SLAB + q * 8, 8)])
    return 0
  lax.fori_loop(0, _SLAB // 8, body, 0)
  @pl.when(s == _NS - 1)
  def _():
    pltpu.sync_copy(zsrc8, acc.at[pl.ds(_SLAB * _NS, 8)])
    pltpu.sync_copy(zsrc8, acc.at[pl.ds(_SLAB * _NS + 8, 8)])
  plsc.subcore_barrier()


def _acc_writeback(acc, out_hbm, c):
  s = lax.axis_index("s")
  plsc.subcore_barrier()
  pltpu.sync_copy(acc.at[pl.ds(s * _SLAB, _SLAB)],
                  out_hbm.at[c, pl.ds(s * _SLAB, _SLAB)])
  @pl.when(s == _NS - 1)
  def _():
    pltpu.sync_copy(acc.at[pl.ds(_SLAB * _NS, 16)],
                    out_hbm.at[c, pl.ds(_SLAB * _NS, 16)])


def _sc_pre_body(vals_hbm, dst_hbm, out_hbm, vbuf, dbuf, acc):
  CH = 400
  c = lax.axis_index("c")
  s = lax.axis_index("s")
  g = c * _NS + s
  _zero_vmem(vbuf, 125, 16)
  _acc_init_and_barrier(acc, vbuf.at[pl.ds(0, 8)])
  def chunk(k, _):
    base = g * _EW + k * CH
    pltpu.sync_copy(vals_hbm.at[pl.ds(base, CH)], vbuf)
    pltpu.sync_copy(dst_hbm.at[pl.ds(base, CH)], dbuf)
    pltpu.sync_copy(vbuf, acc.at[dbuf], add=True)
    return 0
  lax.fori_loop(0, _EW // CH, chunk, 0)
  _acc_writeback(acc, out_hbm, c)


def _run_sc_pre(ealpha16, dst):
  k = pl.kernel(
      _sc_pre_body,
      out_type=jax.ShapeDtypeStruct((_NC, _N, 16), jnp.float32),
      mesh=_mesh(),
      compiler_params=pltpu.CompilerParams(needs_layout_passes=False, use_tc_tiling_on_sc=False),
      scratch_types=[
          pltpu.VMEM((400, 16), jnp.float32),
          pltpu.VMEM((400,), jnp.int32),
          pltpu.VMEM_SHARED((_N, 16), jnp.float32),
      ],
  )
  return k(ealpha16, dst)


def _sc_small_body(heads, wt, off, xl_hbm, src_hbm, dst_hbm, eal_hbm, att_hbm,
                   out_hbm, xlt, sbuf, dbuf, ebuf, stage, attv, acc):
  CH = 400
  c = lax.axis_index("c")
  s = lax.axis_index("s")
  g = c * _NS + s
  _zero_vmem(stage, CH, 8)
  _acc_init_and_barrier(acc, stage.at[pl.ds(0, 8)])
  pltpu.sync_copy(xl_hbm, xlt)
  pltpu.sync_copy(att_hbm, attv)
  asc = [plsc.load_gather(attv, [_full16(hh)]) for hh in range(heads)]
  adc = [plsc.load_gather(attv, [_full16(8 + hh)]) for hh in range(heads)]

  def chunk(k, _):
    base = g * _EW + k * CH
    pltpu.sync_copy(src_hbm.at[pl.ds(base, CH)], sbuf)
    pltpu.sync_copy(dst_hbm.at[pl.ds(base, CH)], dbuf)
    pltpu.sync_copy(eal_hbm.at[pl.ds(base, CH)], ebuf)
    def grp(j, _):
      l16 = j * 16 + _iota16()
      s16 = sbuf[pl.ds(j * 16, 16)]
      d16 = dbuf[pl.ds(j * 16, 16)]
      for hh in range(heads):
        gs = plsc.load_gather(xlt, [s16 * wt + hh])
        gd = plsc.load_gather(xlt, [d16 * wt + hh])
        ge = plsc.load_gather(ebuf, [l16, _full16(off + hh)])
        w = _lrelu_exp(gs * asc[hh] + gd * adc[hh] + ge)
        plsc.store_scatter(stage, [l16, _full16(hh)], w)
        plsc.store_scatter(stage, [l16, _full16(4 + hh)], w * gs)
      return 0
    lax.fori_loop(0, CH // 16, grp, 0)
    pltpu.sync_copy(stage, acc.at[dbuf], add=True)
    return 0
  lax.fori_loop(0, _EW // CH, chunk, 0)
  _acc_writeback(acc, out_hbm, c)


def _run_sc_small(i, xl, src, dst, ealpha16, a_src, a_dst):
  heads = _HEADS[i]
  wt = xl.shape[1]
  att = jnp.zeros((16,), jnp.float32)
  att = att.at[:heads].set(a_src)
  att = att.at[8:8 + heads].set(a_dst)
  body = functools.partial(_sc_small_body, heads, wt, _OFFS[i])
  k = pl.kernel(
      body,
      out_type=jax.ShapeDtypeStruct((_NC, _N, 8), jnp.float32),
      mesh=_mesh(),
      compiler_params=pltpu.CompilerParams(needs_layout_passes=False, use_tc_tiling_on_sc=False),
      scratch_types=[
          pltpu.VMEM((_N * wt,), jnp.float32),
          pltpu.VMEM((400,), jnp.int32),
          pltpu.VMEM((400,), jnp.int32),
          pltpu.VMEM((400, 16), jnp.float32),
          pltpu.VMEM((400, 8), jnp.float32),
          pltpu.VMEM((16,), jnp.float32),
          pltpu.VMEM_SHARED((_N, 8), jnp.float32),
      ],
  )
  return k(xl.reshape(-1), src, dst, ealpha16, att)


def _sc_l5_body(t5_hbm, ad8_hbm, src_hbm, dst_hbm, eal_hbm, out_hbm,
                sbuf, dbuf, ebuf, rows, drows, stage, wbuf, acc, sem):
  CH = 80
  c = lax.axis_index("c")
  s = lax.axis_index("s")
  g = c * _NS + s
  _zero_vmem(stage, CH, 136)
  _acc_init_and_barrier(acc, stage.at[pl.ds(0, 8)])

  def chunk(k, _):
    base = g * _EW + k * CH
    pltpu.sync_copy(src_hbm.at[pl.ds(base, CH)], sbuf)
    pltpu.sync_copy(dst_hbm.at[pl.ds(base, CH)], dbuf)
    pltpu.sync_copy(eal_hbm.at[pl.ds(base, CH)], ebuf)
    pltpu.async_copy(t5_hbm.at[sbuf], rows, sem).wait()
    pltpu.async_copy(ad8_hbm.at[dbuf], drows, sem).wait()
    def grp(j, _):
      l16 = j * 16 + _iota16()
      a_s = plsc.load_gather(rows, [l16, _full16(128)])
      a_d = plsc.load_gather(drows, [l16, _full16(0)])
      ge = plsc.load_gather(ebuf, [l16, _full16(14)])
      w = _lrelu_exp(a_s + a_d + ge)
      wbuf[pl.ds(j * 16, 16)] = w
      plsc.store_scatter(stage, [l16, _full16(128)], w)
      return 0
    lax.fori_loop(0, CH // 16, grp, 0)
    def edge(q, _):
      for u in range(4):
        e = q * 4 + u
        w16 = plsc.load_gather(wbuf, [_full16(e)])
        e16 = _full16(e)
        for kc in range(8):
          col = kc * 16 + _iota16()
          v = plsc.load_gather(rows, [e16, col])
          plsc.store_scatter(stage, [e16, col], w16 * v)
      return 0
    lax.fori_loop(0, CH // 4, edge, 0)
    pltpu.sync_copy(stage, acc.at[dbuf], add=True)
    return 0
  lax.fori_loop(0, _EW // CH, chunk, 0)
  _acc_writeback(acc, out_hbm, c)


def _run_sc_l5(t5, ad8, src, dst, ealpha16):
  k = pl.kernel(
      _sc_l5_body,
      out_type=jax.ShapeDtypeStruct((_NC, _N, 136), jnp.float32),
      mesh=_mesh(),
      compiler_params=pltpu.CompilerParams(needs_layout_passes=False, use_tc_tiling_on_sc=False),
      scratch_types=[
          pltpu.VMEM((80,), jnp.int32),
          pltpu.VMEM((80,), jnp.int32),
          pltpu.VMEM((80, 16), jnp.float32),
          pltpu.VMEM((80, 136), jnp.float32),
          pltpu.VMEM((80, 8), jnp.float32),
          pltpu.VMEM((80, 136), jnp.float32),
          pltpu.VMEM((80,), jnp.float32),
          pltpu.VMEM_SHARED((_N, 136), jnp.float32),
          pltpu.SemaphoreType.DMA,
      ],
  )
  return k(t5, ad8, src, dst, ealpha16)


def kernel(x, edge_index, edge_attr, params):
  gats = params["gats"]
  bns = params["bns"]
  src = edge_index[0]
  dst = edge_index[1]

  chs = (1, 1, 1, 1, 1, 128)
  wcols = []
  for l in range(6):
    p = gats[l]
    h, ch = _HEADS[l], chs[l]
    wer = p["W_e"].reshape(16, h, ch)
    wcols.append(jnp.einsum("dhc,hc->dh", wer, p["att_edge"]))
  wcomb = jnp.concatenate(wcols, axis=1)
  wcomb = jnp.concatenate([wcomb, jnp.zeros((16, 1), jnp.float32)], axis=1)

  ealpha16 = _run_edge_pre(edge_attr, wcomb)
  lacc = _run_sc_pre(ealpha16, dst)
  xl, le = _run_d0(x, gats[0]["W"], lacc)

  for i in range(5):
    p = gats[i]
    a_src = p["att_src"][:, 0]
    a_dst = p["att_dst"][:, 0]
    acc = _run_sc_small(i, xl, src, dst, ealpha16, a_src, a_dst)
    if i < 4:
      xl = _run_mid(i, acc, xl, le, a_src, a_dst, gats[i + 1]["W"],
                    p["bias"],
                    bns[i]["gamma"] if i < 3 else None,
                    bns[i]["beta"] if i < 3 else None)
    else:
      p5 = gats[5]
      sa5 = jnp.sum(p5["W"][0] * p5["att_src"][0])
      sd5 = jnp.sum(p5["W"][0] * p5["att_dst"][0])
      t5, ad8 = _run_d5in(acc, xl, le, a_src, a_dst, p5["W"], p["bias"],
                          sa5, sd5)

  acc5 = _run_sc_l5(t5, ad8, src, dst, ealpha16)
  return _run_final(acc5, t5, ad8, le, gats[5]["bias"])
